# Initial kernel scaffold; baseline (speedup 1.0000x reference)
import functools

import jax
import jax.numpy as jnp
import numpy as np
from jax import lax
from jax.experimental import pallas as pl
from jax.experimental.pallas import tpu as pltpu
from jax.experimental.pallas import tpu_sc as plsc

B = 4
N = 16384
R = 76
PRE = 9000
PAD = 9216
BLK = 256
NBLK = PAD // BLK
POST = 512
THRESH = 0.85
LOC_SCOPE = 3.0
LOC_BIN_SIZE = 0.5
NUM_HEAD_BIN = 12
MEAN_H = 1.52563191462
MEAN_W = 1.62856739989
MEAN_L = 3.88311640418
NW = 32
ROWS_PER_W = (B * PAD) // NW
GCHUNK = 128
NCHUNK = ROWS_PER_W // GCHUNK


def _sort_body(s_ref, ss_ref, gi_ref):
    s = s_ref[...]
    r_io = lax.broadcasted_iota(jnp.int32, (B, 128, 128), 1)
    c_io = lax.broadcasted_iota(jnp.int32, (B, 128, 128), 2)
    vidx = r_io * 128 + c_io
    for k in range(1, 15):
        for j in range(k - 1, -1, -1):
            d = 1 << j
            if j < 7:
                axis = 2
                bit = (c_io >> j) & 1
            else:
                axis = 1
                bit = (r_io >> (j - 7)) & 1
            is_low = bit == 0
            ps = jnp.where(is_low, jnp.roll(s, -d, axis), jnp.roll(s, d, axis))
            pi = jnp.where(is_low, jnp.roll(vidx, -d, axis), jnp.roll(vidx, d, axis))
            f = (s > ps) | ((s == ps) & (vidx < pi))
            if k < 7:
                dir_asc = ((c_io >> k) & 1) == 0
            else:
                dir_asc = ((r_io >> (k - 7)) & 1) == 0
            keep_own = jnp.where(dir_asc, f == is_low, f != is_low)
            s = jnp.where(keep_own, s, ps)
            vidx = jnp.where(keep_own, vidx, pi)
    ss_ref[...] = s[:, :PAD // 128, :]
    gi = vidx + lax.broadcasted_iota(jnp.int32, (B, 128, 128), 0) * N
    gi_ref[...] = gi[:, :PAD // 128, :]


def _sort_topk(scores):
    s3 = scores.reshape(B, 128, 128)
    return pl.pallas_call(
        _sort_body,
        out_shape=(
            jax.ShapeDtypeStruct((B, PAD // 128, 128), jnp.float32),
            jax.ShapeDtypeStruct((B, PAD // 128, 128), jnp.int32),
        ),
    )(s3)


def _sc_gather(table, gidx):
    idx3 = gidx.reshape(NW, NCHUNK, GCHUNK)
    mesh = plsc.VectorSubcoreMesh(core_axis_name="c", subcore_axis_name="s")

    @functools.partial(
        pl.kernel,
        mesh=mesh,
        out_type=jax.ShapeDtypeStruct((NW, NCHUNK, GCHUNK, 80), jnp.float32),
        scratch_types=[
            pltpu.VMEM((NCHUNK, GCHUNK), jnp.int32),
            pltpu.VMEM((NCHUNK, GCHUNK, 80), jnp.float32),
            pltpu.SemaphoreType.DMA,
        ],
    )
    def gather_k(table_hbm, idx_hbm, out_hbm, idx_v, rows_v, sem):
        wid = lax.axis_index("s") * 2 + lax.axis_index("c")
        pltpu.sync_copy(idx_hbm.at[wid], idx_v)
        copies = [
            pltpu.async_copy(table_hbm.at[idx_v.at[j]], rows_v.at[j], sem)
            for j in range(NCHUNK)
        ]
        for c in copies:
            c.wait()
        pltpu.sync_copy(rows_v, out_hbm.at[wid])

    return gather_k(table, idx3).reshape(B * PAD, 80)


def _pair_iou(ax1, ay1, ax2, ay2, aarea, bx1, by1, bx2, by2, barea):
    xx1 = jnp.maximum(ax1, bx1)
    yy1 = jnp.maximum(ay1, by1)
    xx2 = jnp.minimum(ax2, bx2)
    yy2 = jnp.minimum(ay2, by2)
    inter = jnp.maximum(xx2 - xx1, 0.0) * jnp.maximum(yy2 - yy1, 0.0)
    return inter / jnp.maximum(aarea + barea - inter, 1e-8)


def _decode_nms_body(feat_ref, sc_ref, out_ref,
                     x1_r, y1_r, x2_r, y2_r, ar_r, dat_r, keep_r, cnt_r):
    F = lambda j: feat_ref[:, j, :]

    def argmax_take(b0, r0):
        best = F(b0)
        res = F(r0)
        bin_ = jnp.zeros((B, PAD), jnp.float32)
        for j in range(1, 12):
            v = F(b0 + j)
            upd = v > best
            best = jnp.where(upd, v, best)
            res = jnp.where(upd, F(r0 + j), res)
            bin_ = jnp.where(upd, jnp.float32(j), bin_)
        return bin_, res

    xb, xres = argmax_take(0, 24)
    zb, zres = argmax_take(12, 36)
    px = xb * LOC_BIN_SIZE + (LOC_BIN_SIZE / 2) - LOC_SCOPE
    pz = zb * LOC_BIN_SIZE + (LOC_BIN_SIZE / 2) - LOC_SCOPE
    px = px + xres * LOC_BIN_SIZE
    pz = pz + zres * LOC_BIN_SIZE
    py = F(77) + F(48)
    ryb, ryres_n = argmax_take(49, 61)
    apc = (2.0 * np.pi) / NUM_HEAD_BIN
    ry = ryb * apc + ryres_n * (apc / 2.0)
    ry = ry % (2.0 * np.pi)
    ry = jnp.where(ry > np.pi, ry - 2.0 * np.pi, ry)
    h = F(73) * MEAN_H + MEAN_H
    w = F(74) * MEAN_W + MEAN_W
    l = F(75) * MEAN_L + MEAN_L
    px = px + F(76)
    pz = pz + F(78)
    yc = py + h / 2.0

    x1 = px - l / 2.0
    y1 = pz - w / 2.0
    x2 = px + l / 2.0
    y2 = pz + w / 2.0
    x1_r[...] = x1
    y1_r[...] = y1
    x2_r[...] = x2
    y2_r[...] = y2
    ar_r[...] = (x2 - x1) * (y2 - y1)
    for i, ch in enumerate((px, yc, pz, h, w, l, ry, sc_ref[...])):
        dat_r[:, i, :] = ch

    keep_r[...] = jnp.zeros((B, PAD), jnp.float32)
    cnt_r[...] = jnp.zeros((B, 128), jnp.float32)

    refs = (x1_r, y1_r, x2_r, y2_r, ar_r)

    def blk_body(b, carry):
        @pl.when(jnp.min(cnt_r[...]) < float(POST))
        def _():
            base = pl.multiple_of(b * BLK, BLK)
            bc = [r[:, pl.ds(base, BLK)] for r in refs]
            bi = [v[:, :, None] for v in bc]
            bj = [v[:, None, :] for v in bc]
            i_io = lax.broadcasted_iota(jnp.int32, (B, BLK, BLK), 1)
            j_io = lax.broadcasted_iota(jnp.int32, (B, BLK, BLK), 2)
            iou = _pair_iou(*bi, *bj)
            Ef = jnp.where((iou > THRESH) & (i_io < j_io), 1.0, 0.0)

            gpos = base + lax.broadcasted_iota(jnp.int32, (B, BLK), 1)
            s0_init = jnp.where(gpos >= PRE, 1.0, 0.0)

            def prev_body(pb, s0c):
                pbase = pl.multiple_of(pb * BLK, BLK)
                pc = [r[:, pl.ds(pbase, BLK)] for r in refs]
                pi_ = [v[:, :, None] for v in pc]
                piou = _pair_iou(*pi_, *bj)
                pk = keep_r[:, pl.ds(pbase, BLK)][:, :, None]
                hit = jnp.max(jnp.where(piou > THRESH, pk, 0.0), axis=1)
                return jnp.maximum(s0c, hit)

            s0 = lax.fori_loop(0, b, prev_body, s0_init)

            def fp_cond(st):
                return st[1]

            def fp_body(st):
                s, _, it = st
                notS = 1.0 - s
                t = jnp.max(Ef * notS[:, :, None], axis=1)
                s_new = jnp.maximum(s0, jnp.where(t > 0.5, 1.0, 0.0))
                return (s_new, jnp.any(s_new != s), it + 1)

            s_fin, _, _ = lax.while_loop(fp_cond, fp_body,
                                         (s0, jnp.bool_(True), jnp.int32(0)))
            kb = 1.0 - s_fin
            keep_r[:, pl.ds(base, BLK)] = kb
            cnt_r[...] = cnt_r[...] + jnp.sum(kb, axis=1, keepdims=True)
        return carry

    lax.fori_loop(0, NBLK, blk_body, 0)

    k2 = keep_r[...].reshape(B, PAD // 128, 128)
    c_a = lax.broadcasted_iota(jnp.int32, (128, 128), 0)
    c_b = lax.broadcasted_iota(jnp.int32, (128, 128), 1)
    Lx = jnp.where(c_a < c_b, 1.0, 0.0)
    pos_in = lax.dot_general(k2, Lx, (((2,), (0,)), ((), ())),
                             preferred_element_type=jnp.float32)
    rowtot = jnp.sum(k2, axis=2)
    r_a = lax.broadcasted_iota(jnp.int32, (PAD // 128, PAD // 128), 0)
    r_b = lax.broadcasted_iota(jnp.int32, (PAD // 128, PAD // 128), 1)
    Lr = jnp.where(r_a < r_b, 1.0, 0.0)
    rowoff = lax.dot_general(rowtot, Lr, (((1,), (0,)), ((), ())),
                             preferred_element_type=jnp.float32)
    pos = pos_in + rowoff[:, :, None]

    out_ref[...] = jnp.zeros((B, POST, 8), jnp.float32)
    CH = 1152
    nch = PAD // CH
    rows_per_ch = CH // 128
    for c in range(nch):
        first_pos = jnp.min(pos[:, c * rows_per_ch, 0])

        @pl.when(first_pos < float(POST))
        def _(c=c):
            posc = pos[:, c * rows_per_ch:(c + 1) * rows_per_ch, :].reshape(B, CH)
            keepc = k2[:, c * rows_per_ch:(c + 1) * rows_per_ch, :].reshape(B, CH)
            p_io = lax.broadcasted_iota(jnp.float32, (B, POST, CH), 1)
            M = jnp.where((posc[:, None, :] == p_io) & (keepc[:, None, :] > 0.5),
                          1.0, 0.0)
            dat = dat_r[:, :, pl.ds(c * CH, CH)]
            contrib = lax.dot_general(M, dat, (((2,), (2,)), ((0,), (0,))),
                                      preferred_element_type=jnp.float32)
            out_ref[...] += contrib


def _decode_nms(feat, sscore):
    return pl.pallas_call(
        _decode_nms_body,
        out_shape=jax.ShapeDtypeStruct((B, POST, 8), jnp.float32),
        scratch_shapes=[
            pltpu.VMEM((B, PAD), jnp.float32),
            pltpu.VMEM((B, PAD), jnp.float32),
            pltpu.VMEM((B, PAD), jnp.float32),
            pltpu.VMEM((B, PAD), jnp.float32),
            pltpu.VMEM((B, PAD), jnp.float32),
            pltpu.VMEM((B, 8, PAD), jnp.float32),
            pltpu.VMEM((B, PAD), jnp.float32),
            pltpu.VMEM((B, 128), jnp.float32),
        ],
    )(feat, sscore)


def kernel(rpn_scores, rpn_reg, xyz):
    table = jnp.concatenate(
        [rpn_reg.reshape(B * N, R), xyz.reshape(B * N, 3),
         jnp.zeros((B * N, 1), jnp.float32)], axis=1)
    ss3, gi3 = _sort_topk(rpn_scores)
    gathered = _sc_gather(table, gi3.reshape(B * PAD))
    feat = gathered.reshape(B, PAD, 80).transpose(0, 2, 1)
    comb = _decode_nms(feat, ss3.reshape(B, PAD))
    return comb[:, :, :7], comb[:, :, 7]


# speedup vs baseline: 591.9178x; 591.9178x over previous
import functools

import jax
import jax.numpy as jnp
import numpy as np
from jax import lax
from jax.experimental import pallas as pl
from jax.experimental.pallas import tpu as pltpu
from jax.experimental.pallas import tpu_sc as plsc

B = 4
N = 16384
R = 76
PRE = 9000
PAD = 9216
BLK = 256
NBLK = PAD // BLK
POST = 512
THRESH = 0.85
LOC_SCOPE = 3.0
LOC_BIN_SIZE = 0.5
NUM_HEAD_BIN = 12
MEAN_H = 1.52563191462
MEAN_W = 1.62856739989
MEAN_L = 3.88311640418
NW = 32
ROWS_PER_W = (B * PAD) // NW
GCHUNK = 128
NCHUNK = ROWS_PER_W // GCHUNK


def _sort_body(s_ref, ss_ref, gi_ref):
    s = s_ref[...]
    r_io = lax.broadcasted_iota(jnp.int32, (B, 128, 128), 1)
    c_io = lax.broadcasted_iota(jnp.int32, (B, 128, 128), 2)
    vidx = r_io * 128 + c_io
    for k in range(1, 15):
        for j in range(k - 1, -1, -1):
            if j < 7:
                axis = 2
                d = 1 << j
                bit = (c_io >> j) & 1
            else:
                axis = 1
                d = 1 << (j - 7)
                bit = (r_io >> (j - 7)) & 1
            is_low = bit == 0
            ps = jnp.where(is_low, jnp.roll(s, -d, axis), jnp.roll(s, d, axis))
            pi = jnp.where(is_low, jnp.roll(vidx, -d, axis), jnp.roll(vidx, d, axis))
            f = (s > ps) | ((s == ps) & (vidx < pi))
            if k < 7:
                dir_asc = ((c_io >> k) & 1) == 0
            else:
                dir_asc = ((r_io >> (k - 7)) & 1) == 0
            keep_own = (f == is_low) == dir_asc
            s = jnp.where(keep_own, s, ps)
            vidx = jnp.where(keep_own, vidx, pi)
    ss_ref[...] = s[:, :PAD // 128, :]
    gi = vidx + lax.broadcasted_iota(jnp.int32, (B, 128, 128), 0) * N
    gi_ref[...] = gi[:, :PAD // 128, :]


def _sort_topk(scores):
    s3 = scores.reshape(B, 128, 128)
    return pl.pallas_call(
        _sort_body,
        out_shape=(
            jax.ShapeDtypeStruct((B, PAD // 128, 128), jnp.float32),
            jax.ShapeDtypeStruct((B, PAD // 128, 128), jnp.int32),
        ),
    )(s3)


def _sc_gather(table, gidx):
    idx3 = gidx.reshape(NW, NCHUNK, GCHUNK)
    mesh = plsc.VectorSubcoreMesh(core_axis_name="c", subcore_axis_name="s")

    @functools.partial(
        pl.kernel,
        mesh=mesh,
        out_type=jax.ShapeDtypeStruct((NW, NCHUNK, GCHUNK, 128), jnp.float32),
        scratch_types=[
            pltpu.VMEM((NCHUNK, GCHUNK), jnp.int32),
            pltpu.VMEM((2, GCHUNK, 128), jnp.float32),
            pltpu.SemaphoreType.DMA,
        ],
    )
    def gather_k(table_hbm, idx_hbm, out_hbm, idx_v, rows_v, sem):
        wid = lax.axis_index("s") * 2 + lax.axis_index("c")
        pltpu.sync_copy(idx_hbm.at[wid], idx_v)
        cp = [None, None]
        for j in range(NCHUNK):
            cp[j & 1] = pltpu.async_copy(
                table_hbm.at[idx_v.at[j]], rows_v.at[j & 1], sem)
            if j >= 1:
                cp[(j - 1) & 1].wait()
                pltpu.sync_copy(rows_v.at[(j - 1) & 1], out_hbm.at[wid, j - 1])
        cp[(NCHUNK - 1) & 1].wait()
        pltpu.sync_copy(rows_v.at[(NCHUNK - 1) & 1], out_hbm.at[wid, NCHUNK - 1])

    return gather_k(table, idx3).reshape(B * PAD, 128)


def _pair_iou(ax1, ay1, ax2, ay2, aarea, bx1, by1, bx2, by2, barea):
    xx1 = jnp.maximum(ax1, bx1)
    yy1 = jnp.maximum(ay1, by1)
    xx2 = jnp.minimum(ax2, bx2)
    yy2 = jnp.minimum(ay2, by2)
    inter = jnp.maximum(xx2 - xx1, 0.0) * jnp.maximum(yy2 - yy1, 0.0)
    return inter / jnp.maximum(aarea + barea - inter, 1e-8)


def _decode_nms_body(feat_ref, sc_ref, out_ref,
                     x1_r, y1_r, x2_r, y2_r, ar_r, dat_r, keep_r, cnt_r):
    F = lambda j: feat_ref[:, j, :]

    def argmax_take(b0, r0):
        best = F(b0)
        res = F(r0)
        bin_ = jnp.zeros((B, PAD), jnp.float32)
        for j in range(1, 12):
            v = F(b0 + j)
            upd = v > best
            best = jnp.where(upd, v, best)
            res = jnp.where(upd, F(r0 + j), res)
            bin_ = jnp.where(upd, jnp.float32(j), bin_)
        return bin_, res

    xb, xres = argmax_take(0, 24)
    zb, zres = argmax_take(12, 36)
    px = xb * LOC_BIN_SIZE + (LOC_BIN_SIZE / 2) - LOC_SCOPE
    pz = zb * LOC_BIN_SIZE + (LOC_BIN_SIZE / 2) - LOC_SCOPE
    px = px + xres * LOC_BIN_SIZE
    pz = pz + zres * LOC_BIN_SIZE
    py = F(77) + F(48)
    ryb, ryres_n = argmax_take(49, 61)
    apc = (2.0 * np.pi) / NUM_HEAD_BIN
    ry = ryb * apc + ryres_n * (apc / 2.0)
    ry = ry % (2.0 * np.pi)
    ry = jnp.where(ry > np.pi, ry - 2.0 * np.pi, ry)
    h = F(73) * MEAN_H + MEAN_H
    w = F(74) * MEAN_W + MEAN_W
    l = F(75) * MEAN_L + MEAN_L
    px = px + F(76)
    pz = pz + F(78)
    yc = py + h / 2.0

    x1 = px - l / 2.0
    y1 = pz - w / 2.0
    x2 = px + l / 2.0
    y2 = pz + w / 2.0
    x1_r[...] = x1
    y1_r[...] = y1
    x2_r[...] = x2
    y2_r[...] = y2
    ar_r[...] = (x2 - x1) * (y2 - y1)
    for i, ch in enumerate((px, yc, pz, h, w, l, ry, sc_ref[...])):
        dat_r[:, i, :] = ch

    keep_r[...] = jnp.zeros((B, PAD), jnp.float32)
    cnt_r[...] = jnp.zeros((B, 128), jnp.float32)

    refs = (x1_r, y1_r, x2_r, y2_r, ar_r)

    def blk_body(b, carry):
        @pl.when(jnp.min(cnt_r[...]) < float(POST))
        def _():
            base = pl.multiple_of(b * BLK, BLK)
            bc = [r[:, pl.ds(base, BLK)] for r in refs]
            bi = [v[:, :, None] for v in bc]
            bj = [v[:, None, :] for v in bc]
            i_io = lax.broadcasted_iota(jnp.int32, (B, BLK, BLK), 1)
            j_io = lax.broadcasted_iota(jnp.int32, (B, BLK, BLK), 2)
            iou = _pair_iou(*bi, *bj)
            Ef = jnp.where((iou > THRESH) & (i_io < j_io), 1.0, 0.0)

            gpos = base + lax.broadcasted_iota(jnp.int32, (B, BLK), 1)
            s0_init = jnp.where(gpos >= PRE, 1.0, 0.0)

            def prev_body(pb, s0c):
                pbase = pl.multiple_of(pb * BLK, BLK)
                pc = [r[:, pl.ds(pbase, BLK)] for r in refs]
                pi_ = [v[:, :, None] for v in pc]
                piou = _pair_iou(*pi_, *bj)
                pk = keep_r[:, pl.ds(pbase, BLK)][:, :, None]
                hit = jnp.max(jnp.where(piou > THRESH, pk, 0.0), axis=1)
                return jnp.maximum(s0c, hit)

            s0 = lax.fori_loop(0, b, prev_body, s0_init)

            def fp_cond(st):
                return st[1]

            def fp_body(st):
                s, _, it = st
                notS = 1.0 - s
                t = jnp.max(Ef * notS[:, :, None], axis=1)
                s_new = jnp.maximum(s0, jnp.where(t > 0.5, 1.0, 0.0))
                return (s_new, jnp.any(s_new != s), it + 1)

            s_fin, _, _ = lax.while_loop(fp_cond, fp_body,
                                         (s0, jnp.bool_(True), jnp.int32(0)))
            kb = 1.0 - s_fin
            keep_r[:, pl.ds(base, BLK)] = kb
            cnt_r[...] = cnt_r[...] + jnp.sum(kb, axis=1, keepdims=True)
        return carry

    lax.fori_loop(0, NBLK, blk_body, 0)

    k2 = keep_r[...].reshape(B, PAD // 128, 128)
    c_a = lax.broadcasted_iota(jnp.int32, (128, 128), 0)
    c_b = lax.broadcasted_iota(jnp.int32, (128, 128), 1)
    Lx = jnp.where(c_a < c_b, 1.0, 0.0)
    pos_in = lax.dot_general(k2, Lx, (((2,), (0,)), ((), ())),
                             preferred_element_type=jnp.float32)
    rowtot = jnp.sum(k2, axis=2)
    r_a = lax.broadcasted_iota(jnp.int32, (PAD // 128, PAD // 128), 0)
    r_b = lax.broadcasted_iota(jnp.int32, (PAD // 128, PAD // 128), 1)
    Lr = jnp.where(r_a < r_b, 1.0, 0.0)
    rowoff = lax.dot_general(rowtot, Lr, (((1,), (0,)), ((), ())),
                             preferred_element_type=jnp.float32)
    pos = pos_in + rowoff[:, :, None]

    out_ref[...] = jnp.zeros((B, POST, 8), jnp.float32)
    CH = 1152
    nch = PAD // CH
    rows_per_ch = CH // 128
    for c in range(nch):
        first_pos = jnp.min(pos[:, c * rows_per_ch, 0])

        @pl.when(first_pos < float(POST))
        def _(c=c):
            posc = pos[:, c * rows_per_ch:(c + 1) * rows_per_ch, :].reshape(B, CH)
            keepc = k2[:, c * rows_per_ch:(c + 1) * rows_per_ch, :].reshape(B, CH)
            p_io = lax.broadcasted_iota(jnp.int32, (B, POST, CH), 1).astype(jnp.float32)
            M = jnp.where((posc[:, None, :] == p_io) & (keepc[:, None, :] > 0.5),
                          1.0, 0.0)
            dat = dat_r[:, :, pl.ds(c * CH, CH)]
            contrib = lax.dot_general(M, dat, (((2,), (2,)), ((0,), (0,))),
                                      preferred_element_type=jnp.float32)
            out_ref[...] += contrib


def _decode_nms(feat, sscore):
    return pl.pallas_call(
        _decode_nms_body,
        out_shape=jax.ShapeDtypeStruct((B, POST, 8), jnp.float32),
        scratch_shapes=[
            pltpu.VMEM((B, PAD), jnp.float32),
            pltpu.VMEM((B, PAD), jnp.float32),
            pltpu.VMEM((B, PAD), jnp.float32),
            pltpu.VMEM((B, PAD), jnp.float32),
            pltpu.VMEM((B, PAD), jnp.float32),
            pltpu.VMEM((B, 8, PAD), jnp.float32),
            pltpu.VMEM((B, PAD), jnp.float32),
            pltpu.VMEM((B, 128), jnp.float32),
        ],
    )(feat, sscore)


def kernel(rpn_scores, rpn_reg, xyz):
    table = jnp.concatenate(
        [rpn_reg.reshape(B * N, R), xyz.reshape(B * N, 3),
         jnp.zeros((B * N, 49), jnp.float32)], axis=1)
    ss3, gi3 = _sort_topk(rpn_scores)
    gathered = _sc_gather(table, gi3.reshape(B * PAD))
    feat = gathered.reshape(B, PAD, 128)[:, :, :80].transpose(0, 2, 1)
    comb = _decode_nms(feat, ss3.reshape(B, PAD))
    return comb[:, :, :7], comb[:, :, 7]
